# baseline (speedup 1.0000x reference)
import functools

import jax
import jax.numpy as jnp
from jax import lax
from jax.experimental import pallas as pl
from jax.experimental.pallas import tpu as pltpu
from jax.experimental.pallas import tpu_sc as plsc

BATCH = 1024
SEQ = 200
WORD_DIM = 128
POS_DIM = 5
NPOS = 400
OUT_DIM = WORD_DIM + 2 * POS_DIM
TOK = BATCH * SEQ

_NW = 32
_PER_W = TOK // _NW
_C = 128
_CHUNKS = _PER_W // _C
_L = 16
_OBW = _C + 1


def _make_kernel():
    mesh = plsc.VectorSubcoreMesh(core_axis_name="c", subcore_axis_name="s")

    @functools.partial(
        pl.kernel,
        mesh=mesh,
        compiler_params=pltpu.CompilerParams(
            needs_layout_passes=False, use_tc_tiling_on_sc=False),
        out_type=jax.ShapeDtypeStruct(
            (OUT_DIM, SEQ // 8, BATCH // 128, 8, 128), jnp.float32),
        scratch_types=[
            pltpu.VMEM((_PER_W,), jnp.int32),
            pltpu.VMEM((_PER_W,), jnp.int32),
            pltpu.VMEM((_PER_W,), jnp.int32),
            pltpu.VMEM((NPOS * POS_DIM,), jnp.float32),
            pltpu.VMEM((NPOS * POS_DIM,), jnp.float32),
            pltpu.VMEM((_C, WORD_DIM), jnp.float32),
            pltpu.VMEM((_C, WORD_DIM), jnp.float32),
            pltpu.VMEM((OUT_DIM, _OBW), jnp.float32),
            pltpu.VMEM((OUT_DIM, _OBW), jnp.float32),
            pltpu.SemaphoreType.DMA,
            pltpu.SemaphoreType.DMA,
            pltpu.SemaphoreType.DMA,
            pltpu.SemaphoreType.DMA,
        ],
    )
    def k(word_hbm, p1_hbm, p2_hbm, wt_hbm, p1t_hbm, p2t_hbm, out_hbm,
          widx, p1idx, p2idx, p1t_v, p2t_v, wbuf0, wbuf1, obuf0, obuf1,
          sem0, sem1, osem0, osem1):
        wid = lax.axis_index("s") * 2 + lax.axis_index("c")
        tbase = wid * _PER_W
        pltpu.sync_copy(p1t_hbm, p1t_v)
        pltpu.sync_copy(p2t_hbm, p2t_v)
        pltpu.sync_copy(word_hbm.at[pl.ds(tbase, _PER_W)], widx)
        pltpu.sync_copy(p1_hbm.at[pl.ds(tbase, _PER_W)], p1idx)
        pltpu.sync_copy(p2_hbm.at[pl.ds(tbase, _PER_W)], p2idx)
        lane = lax.iota(jnp.int32, _L)
        dvecs = [c * _L + lane for c in range(WORD_DIM // _L)]

        def gather(c, buf, sem):
            return pltpu.async_copy(
                wt_hbm.at[widx.at[pl.ds(c * _C, _C)]], buf, sem)

        def do_chunk(c, buf, sem, obuf, osem):
            j0 = tbase + c * _C
            s = j0 // BATCH
            b0 = pl.multiple_of(j0 % BATCH, _C)
            dst = out_hbm.at[:, s // 8, b0 // _C, s % 8, :]

            @pl.when(c >= 2)
            def _():
                pltpu.make_async_copy(obuf.at[:, pl.ds(0, _C)], dst, osem).wait()

            @plsc.parallel_loop(0, _C // _L, unroll=4)
            def pgroup(g):
                i1 = p1idx[pl.ds(c * _C + g * _L, _L)] * POS_DIM
                i2 = p2idx[pl.ds(c * _C + g * _L, _L)] * POS_DIM
                for dd in range(POS_DIM):
                    obuf[WORD_DIM + dd, pl.ds(g * _L, _L)] = (
                        plsc.load_gather(p1t_v, [i1 + dd]))
                    obuf[WORD_DIM + POS_DIM + dd, pl.ds(g * _L, _L)] = (
                        plsc.load_gather(p2t_v, [i2 + dd]))
            pltpu.make_async_copy(
                wt_hbm.at[widx.at[pl.ds(c * _C, _C)]], buf, sem).wait()

            @plsc.parallel_loop(0, _C, unroll=8)
            def trow(t):
                col = jnp.full((_L,), t, jnp.int32)
                for cc in range(WORD_DIM // _L):
                    v = buf[t, pl.ds(cc * _L, _L)]
                    plsc.store_scatter(obuf, [dvecs[cc], col], v)
            pltpu.async_copy(obuf.at[:, pl.ds(0, _C)], dst, osem)

        gather(0, wbuf0, sem0)

        def pair(p, carry):
            c0 = p * 2
            gather(c0 + 1, wbuf1, sem1)
            do_chunk(c0, wbuf0, sem0, obuf0, osem0)

            @pl.when(c0 + 2 < _CHUNKS)
            def _():
                gather(c0 + 2, wbuf0, sem0)

            do_chunk(c0 + 1, wbuf1, sem1, obuf1, osem1)
            return carry

        lax.fori_loop(0, _CHUNKS // 2, pair, 0)
        pltpu.make_async_copy(
            obuf0.at[:, pl.ds(0, _C)],
            out_hbm.at[:, 0, 0, 0, :], osem0).wait()
        pltpu.make_async_copy(
            obuf1.at[:, pl.ds(0, _C)],
            out_hbm.at[:, 0, 0, 0, :], osem1).wait()

    return k


_k = _make_kernel()


def kernel(word, pos1, pos2, word_table, pos1_table, pos2_table):
    p1t = pos1_table.at[0].set(0.0)
    p2t = pos2_table.at[0].set(0.0)
    out_p = _k(word.T.reshape(TOK).astype(jnp.int32),
               pos1.T.reshape(TOK).astype(jnp.int32),
               pos2.T.reshape(TOK).astype(jnp.int32),
               word_table, p1t.reshape(-1), p2t.reshape(-1))
    return jnp.transpose(out_p, (2, 4, 1, 3, 0)).reshape(BATCH, SEQ, OUT_DIM)


# speedup vs baseline: 1.0896x; 1.0896x over previous
import functools

import jax
import jax.numpy as jnp
from jax import lax
from jax.experimental import pallas as pl
from jax.experimental.pallas import tpu as pltpu
from jax.experimental.pallas import tpu_sc as plsc

BATCH = 1024
SEQ = 200
WORD_DIM = 128
POS_DIM = 5
NPOS = 400
OUT_DIM = WORD_DIM + 2 * POS_DIM
TOK = BATCH * SEQ

_NW = 32
_PER_W = TOK // _NW
_C = 128
_CHUNKS = _PER_W // _C
_L = 16
_OBW = _C + 1


def _make_kernel():
    mesh = plsc.VectorSubcoreMesh(core_axis_name="c", subcore_axis_name="s")

    @functools.partial(
        pl.kernel,
        mesh=mesh,
        compiler_params=pltpu.CompilerParams(
            needs_layout_passes=False, use_tc_tiling_on_sc=False),
        out_type=jax.ShapeDtypeStruct(
            (OUT_DIM, SEQ // 8, BATCH // 128, 8, 128), jnp.float32),
        scratch_types=[
            pltpu.VMEM((_PER_W,), jnp.int32),
            pltpu.VMEM((_PER_W,), jnp.int32),
            pltpu.VMEM((_PER_W,), jnp.int32),
            pltpu.VMEM((NPOS * POS_DIM,), jnp.float32),
            pltpu.VMEM((NPOS * POS_DIM,), jnp.float32),
            pltpu.VMEM((_C, WORD_DIM), jnp.float32),
            pltpu.VMEM((_C, WORD_DIM), jnp.float32),
            pltpu.VMEM((OUT_DIM, _OBW), jnp.float32),
            pltpu.VMEM((OUT_DIM, _OBW), jnp.float32),
            pltpu.SemaphoreType.DMA,
            pltpu.SemaphoreType.DMA,
            pltpu.SemaphoreType.DMA,
            pltpu.SemaphoreType.DMA,
        ],
    )
    def k(word_hbm, p1_hbm, p2_hbm, wt_hbm, p1t_hbm, p2t_hbm, out_hbm,
          widx, p1idx, p2idx, p1t_v, p2t_v, wbuf0, wbuf1, obuf0, obuf1,
          sem0, sem1, osem0, osem1):
        wid = lax.axis_index("s") * 2 + lax.axis_index("c")
        tbase = wid * _PER_W
        pltpu.sync_copy(p1t_hbm, p1t_v)
        pltpu.sync_copy(p2t_hbm, p2t_v)
        pltpu.sync_copy(word_hbm.at[pl.ds(tbase, _PER_W)], widx)
        pltpu.sync_copy(p1_hbm.at[pl.ds(tbase, _PER_W)], p1idx)
        pltpu.sync_copy(p2_hbm.at[pl.ds(tbase, _PER_W)], p2idx)
        lane = lax.iota(jnp.int32, _L)
        dvecs = [c * _L + lane for c in range(WORD_DIM // _L)]

        def gather(c, buf, sem):
            return pltpu.async_copy(
                wt_hbm.at[widx.at[pl.ds(c * _C, _C)]], buf, sem)

        def do_chunk(c, buf, sem, obuf, osem):
            j0 = tbase + c * _C
            s = j0 // BATCH
            b0 = pl.multiple_of(j0 % BATCH, _C)
            dst = out_hbm.at[:, s // 8, b0 // _C, s % 8, :]

            @pl.when(c >= 2)
            def _():
                pltpu.make_async_copy(obuf.at[:, pl.ds(0, _C)], dst, osem).wait()

            @plsc.parallel_loop(0, _C // _L, unroll=2)
            def pgroup(g):
                i1 = p1idx[pl.ds(c * _C + g * _L, _L)] * POS_DIM
                i2 = p2idx[pl.ds(c * _C + g * _L, _L)] * POS_DIM
                for dd in range(POS_DIM):
                    obuf[WORD_DIM + dd, pl.ds(g * _L, _L)] = (
                        plsc.load_gather(p1t_v, [i1 + dd]))
                    obuf[WORD_DIM + POS_DIM + dd, pl.ds(g * _L, _L)] = (
                        plsc.load_gather(p2t_v, [i2 + dd]))
            pltpu.make_async_copy(
                wt_hbm.at[widx.at[pl.ds(c * _C, _C)]], buf, sem).wait()

            @plsc.parallel_loop(0, _C, unroll=4)
            def trow(t):
                col = jnp.full((_L,), t, jnp.int32)
                for cc in range(WORD_DIM // _L):
                    v = buf[t, pl.ds(cc * _L, _L)]
                    plsc.store_scatter(obuf, [dvecs[cc], col], v)
            pltpu.async_copy(obuf.at[:, pl.ds(0, _C)], dst, osem)

        gather(0, wbuf0, sem0)

        def pair(p, carry):
            c0 = p * 2
            gather(c0 + 1, wbuf1, sem1)
            do_chunk(c0, wbuf0, sem0, obuf0, osem0)

            @pl.when(c0 + 2 < _CHUNKS)
            def _():
                gather(c0 + 2, wbuf0, sem0)

            do_chunk(c0 + 1, wbuf1, sem1, obuf1, osem1)
            return carry

        lax.fori_loop(0, _CHUNKS // 2, pair, 0)
        pltpu.make_async_copy(
            obuf0.at[:, pl.ds(0, _C)],
            out_hbm.at[:, 0, 0, 0, :], osem0).wait()
        pltpu.make_async_copy(
            obuf1.at[:, pl.ds(0, _C)],
            out_hbm.at[:, 0, 0, 0, :], osem1).wait()

    return k


_k = _make_kernel()


def kernel(word, pos1, pos2, word_table, pos1_table, pos2_table):
    p1t = pos1_table.at[0].set(0.0)
    p2t = pos2_table.at[0].set(0.0)
    out_p = _k(word.T.reshape(TOK).astype(jnp.int32),
               pos1.T.reshape(TOK).astype(jnp.int32),
               pos2.T.reshape(TOK).astype(jnp.int32),
               word_table, p1t.reshape(-1), p2t.reshape(-1))
    return jnp.transpose(out_p, (2, 4, 1, 3, 0)).reshape(BATCH, SEQ, OUT_DIM)
